# baseline (speedup 1.0000x reference)
import functools
import jax
import jax.numpy as jnp
from jax import lax
from jax.experimental import pallas as pl
from jax.experimental.pallas import tpu as pltpu
from jax.experimental.pallas import tpu_sc as plsc

NC, NS = 2, 16
NW = NC * NS
CS = 16
NSLOT = 4
UNROLL = 8


def kernel(x, pe):
    B, S, D = x.shape
    xf = x.reshape(B * S * D)
    pef = pe.reshape(-1)
    s_per_w = S // NW
    n_sub = s_per_w // CS
    n_chunks = n_sub * B
    chunk = CS * D

    mesh = plsc.VectorSubcoreMesh(core_axis_name="c", subcore_axis_name="s")

    @functools.partial(
        pl.kernel,
        mesh=mesh,
        out_type=jax.ShapeDtypeStruct((B * S * D,), jnp.float32),
        scratch_types=(
            [pltpu.VMEM((chunk,), jnp.float32) for _ in range(NSLOT + 2)]
            + [pltpu.SemaphoreType.DMA for _ in range(NSLOT * 2 + 2)]
        ),
    )
    def k(x_hbm, pe_hbm, out_hbm, *rest):
        bxs = rest[:NSLOT]
        bps = rest[NSLOT:NSLOT + 2]
        sems = rest[NSLOT + 2:]
        sis, sos, sps = sems[:NSLOT], sems[NSLOT:2 * NSLOT], sems[2 * NSLOT:]
        sid = lax.axis_index("s")
        wid = sid * NC + lax.axis_index("c")
        s_base = wid * s_per_w

        def off0(i):
            j, b = i // B, i % B
            return (b * S + s_base + j * CS) * D

        pend_in = {}
        pend_out = {}
        pend_pe = {}

        def issue_in(i):
            slot = i % NSLOT
            if i - NSLOT in pend_out:
                pend_out.pop(i - NSLOT).wait()
            pend_in[i] = pltpu.async_copy(
                x_hbm.at[pl.ds(off0(i), chunk)], bxs[slot], sis[slot])

        def issue_pe(j):
            ps = j % 2
            pend_pe[j] = pltpu.async_copy(
                pe_hbm.at[pl.ds((s_base + j * CS) * D, chunk)],
                bps[ps], sps[ps])

        issue_pe(0)
        issue_in(0)
        issue_in(1)
        for i in range(n_chunks):
            j, b = i // B, i % B
            slot, ps = i % NSLOT, j % 2
            if b == 0:
                pend_pe.pop(j).wait()
            pend_in.pop(i).wait()
            bx, bp = bxs[slot], bps[ps]

            @plsc.parallel_loop(0, chunk, step=16, unroll=UNROLL)
            def _(o):
                sl = pl.ds(o, 16)
                bx[sl] = bx[sl] + bp[sl]

            pend_out[i] = pltpu.async_copy(
                bx, out_hbm.at[pl.ds(off0(i), chunk)], sos[slot])
            if b == B - 1 and j + 1 < n_sub:
                issue_pe(j + 1)
            if i + 2 < n_chunks:
                issue_in(i + 2)
        for i in sorted(pend_out):
            pend_out.pop(i).wait()

    out = k(xf, pef)
    return out.reshape(B, S, D)


# speedup vs baseline: 1.0038x; 1.0038x over previous
import functools
import jax
import jax.numpy as jnp
from jax import lax
from jax.experimental import pallas as pl
from jax.experimental.pallas import tpu as pltpu
from jax.experimental.pallas import tpu_sc as plsc

NC, NS = 2, 16
NW = NC * NS
CS = 16
NSLOT = 4
UNROLL = 8


def kernel(x, pe):
    B, S, D = x.shape
    xf = x.reshape(B * S * D)
    pef = pe.reshape(-1)
    s_per_w = S // NW
    n_sub = s_per_w // CS
    n_chunks = n_sub * B
    chunk = CS * D

    mesh = plsc.VectorSubcoreMesh(core_axis_name="c", subcore_axis_name="s")

    @functools.partial(
        pl.kernel,
        mesh=mesh,
        out_type=jax.ShapeDtypeStruct((B * S * D,), jnp.float32),
        scratch_types=(
            [pltpu.VMEM((chunk,), jnp.float32) for _ in range(NSLOT + 2)]
            + [pltpu.SemaphoreType.DMA for _ in range(NSLOT * 2 + 2)]
        ),
    )
    def k(x_hbm, pe_hbm, out_hbm, *rest):
        bxs = rest[:NSLOT]
        bps = rest[NSLOT:NSLOT + 2]
        sems = rest[NSLOT + 2:]
        sis, sos, sps = sems[:NSLOT], sems[NSLOT:2 * NSLOT], sems[2 * NSLOT:]
        sid = lax.axis_index("s")
        wid = sid * NC + lax.axis_index("c")
        s_base = wid * s_per_w

        def off0(i):
            j, b = i // B, i % B
            return (b * S + s_base + j * CS) * D

        pend_in = {}
        pend_out = {}
        pend_pe = {}

        def issue_in(i):
            slot = i % NSLOT
            if i - NSLOT in pend_out:
                pend_out.pop(i - NSLOT).wait()
            pend_in[i] = pltpu.async_copy(
                x_hbm.at[pl.ds(off0(i), chunk)], bxs[slot], sis[slot])

        def issue_pe(j):
            ps = j % 2
            pend_pe[j] = pltpu.async_copy(
                pe_hbm.at[pl.ds((s_base + j * CS) * D, chunk)],
                bps[ps], sps[ps])

        issue_pe(0)
        issue_in(0)
        issue_in(1)
        for i in range(n_chunks):
            j, b = i // B, i % B
            slot, ps = i % NSLOT, j % 2
            if b == 0:
                pend_pe.pop(j).wait()
            pend_in.pop(i).wait()
            bx, bp = bxs[slot], bps[ps]

            @plsc.parallel_loop(0, chunk, step=16, unroll=UNROLL)
            def _(o):
                sl = pl.ds(o, 16)
                plsc.addupdate(bx.at[sl], bp[sl])

            pend_out[i] = pltpu.async_copy(
                bx, out_hbm.at[pl.ds(off0(i), chunk)], sos[slot])
            if b == B - 1 and j + 1 < n_sub:
                issue_pe(j + 1)
            if i + 2 < n_chunks:
                issue_in(i + 2)
        for i in sorted(pend_out):
            pend_out.pop(i).wait()

    out = k(xf, pef)
    return out.reshape(B, S, D)


# speedup vs baseline: 2.9684x; 2.9571x over previous
import functools
import jax
import jax.numpy as jnp
from jax import lax
from jax.experimental import pallas as pl
from jax.experimental.pallas import tpu as pltpu
from jax.experimental.pallas import tpu_sc as plsc

NC, NS = 2, 16
NW = NC * NS
CS = 16
NSLOT = 5
UNROLL = 8


def kernel(x, pe):
    B, S, D = x.shape
    xf = x.reshape(B * S, D)
    s_per_w = S // NW
    n_sub = s_per_w // CS
    n_chunks = n_sub * B
    chunk = CS * D

    mesh = plsc.VectorSubcoreMesh(core_axis_name="c", subcore_axis_name="s")

    @functools.partial(
        pl.kernel,
        mesh=mesh,
        out_type=jax.ShapeDtypeStruct((B * S, D), jnp.float32),
        compiler_params=pltpu.CompilerParams(use_tc_tiling_on_sc=True),
        scratch_types=(
            [pltpu.VMEM((CS, D), jnp.float32) for _ in range(NSLOT + 2)]
            + [pltpu.SemaphoreType.DMA for _ in range(NSLOT * 2 + 2)]
        ),
    )
    def k(x_hbm, pe_hbm, out_hbm, *rest):
        bxs = rest[:NSLOT]
        bps = rest[NSLOT:NSLOT + 2]
        sems = rest[NSLOT + 2:]
        sis, sos, sps = sems[:NSLOT], sems[NSLOT:2 * NSLOT], sems[2 * NSLOT:]
        sid = lax.axis_index("s")
        wid = sid * NC + lax.axis_index("c")
        s_base = wid * s_per_w

        def row0(i):
            j, b = i // B, i % B
            return b * S + s_base + j * CS

        pend_in = {}
        pend_out = {}
        pend_pe = {}

        def issue_in(i):
            slot = i % NSLOT
            if i - NSLOT in pend_out:
                pend_out.pop(i - NSLOT).wait()
            pend_in[i] = pltpu.async_copy(
                x_hbm.at[pl.ds(row0(i), CS)], bxs[slot], sis[slot])

        def issue_pe(j):
            ps = j % 2
            pend_pe[j] = pltpu.async_copy(
                pe_hbm.at[pl.ds(s_base + j * CS, CS)], bps[ps], sps[ps])

        issue_pe(0)
        issue_pe(1)
        issue_in(0)
        issue_in(1)
        issue_in(2)
        for i in range(n_chunks):
            j, b = i // B, i % B
            slot, ps = i % NSLOT, j % 2
            if b == 0:
                pend_pe.pop(j).wait()
            pend_in.pop(i).wait()
            bx, bp = bxs[slot], bps[ps]

            @plsc.parallel_loop(0, CS, step=1, unroll=2)
            def _(r):
                @plsc.parallel_loop(0, D, step=16, unroll=8)
                def _(c):
                    plsc.addupdate(bx.at[r, pl.ds(c, 16)], bp[r, pl.ds(c, 16)])

            pend_out[i] = pltpu.async_copy(
                bx, out_hbm.at[pl.ds(row0(i), CS)], sos[slot])
            if b == B - 1 and j + 2 < n_sub:
                issue_pe(j + 2)
            if i + 3 < n_chunks:
                issue_in(i + 3)
        for i in sorted(pend_out):
            pend_out.pop(i).wait()

    out = k(xf, pe)
    return out.reshape(B, S, D)
